# baseline (speedup 1.0000x reference)
import functools
import math

import jax
import jax.numpy as jnp
import numpy as np
from jax import lax
from jax.experimental import pallas as pl
from jax.experimental.pallas import tpu as pltpu
from jax.experimental.pallas import tpu_sc as plsc

_SMOOTHING = 0.1
_BN = 32
_NC = 2
_NS = 16
_LANES = 16


def _gather_body(v, pred_flat_ref, tgt_ref, out_ref, idx_v, vals_v, sem):
    bpw = idx_v.shape[0]
    wid = lax.axis_index("s") * _NC + lax.axis_index("c")
    base = wid * bpw
    pltpu.sync_copy(tgt_ref.at[pl.ds(base, bpw)], idx_v)
    for j in range(bpw // _LANES):
        tv = idx_v[pl.ds(j * _LANES, _LANES)]
        rows = (base + j * _LANES) + lax.iota(jnp.int32, _LANES)
        idx_v[pl.ds(j * _LANES, _LANES)] = rows * v + tv
    pltpu.async_copy(pred_flat_ref.at[idx_v], vals_v, sem).wait()
    pltpu.sync_copy(vals_v, out_ref.at[pl.ds(base, bpw)])


def _sc_gather(pred_flat, target, v):
    n = target.shape[0]
    nw = _NC * _NS
    bpw = n // nw
    mesh = plsc.VectorSubcoreMesh(core_axis_name="c", subcore_axis_name="s")
    return pl.kernel(
        functools.partial(_gather_body, v),
        out_type=jax.ShapeDtypeStruct((n,), jnp.float32),
        mesh=mesh,
        scratch_types=[
            pltpu.VMEM((bpw,), jnp.int32),
            pltpu.VMEM((bpw,), jnp.float32),
            pltpu.SemaphoreType.DMA,
        ],
    )(pred_flat, target)


def _loss_body(eps, coef_g, c_row, tgt_ref, g_ref, pred_ref, out_ref):
    i = pl.program_id(0)
    t = tgt_ref[...]
    g = g_ref[...]
    x = pred_ref[...]
    valid = t != 0
    s = jnp.sum(x, axis=1, keepdims=True) - x[:, 0:1]
    part = jnp.sum(jnp.where(valid, s, 0.0))
    gpart = jnp.sum(jnp.where(valid, g, 0.0))
    cnt = jnp.sum(jnp.where(valid, 1.0, 0.0))

    @pl.when(i == 0)
    def _():
        out_ref[0, 0] = 0.0

    out_ref[0, 0] += c_row * cnt - eps * part - coef_g * gpart


def kernel(pred, target):
    n, v = pred.shape
    eps = _SMOOTHING / (v - 2)
    eps32 = float(np.float32(eps))
    c_row = (v - 2) * (eps32 * math.log(eps32)) + 0.9 * math.log(0.9)
    coef_g = (1.0 - _SMOOTHING) - eps

    g = _sc_gather(pred.reshape(-1), target, v)

    tgt2d = target.reshape(n, 1)
    g2d = g.reshape(n, 1)
    grid = (n // _BN,)
    out = pl.pallas_call(
        functools.partial(_loss_body, eps, coef_g, c_row),
        grid=grid,
        in_specs=[
            pl.BlockSpec((_BN, 1), lambda i: (i, 0)),
            pl.BlockSpec((_BN, 1), lambda i: (i, 0)),
            pl.BlockSpec((_BN, v), lambda i: (i, 0)),
        ],
        out_specs=pl.BlockSpec(
            (1, 1), lambda i: (0, 0), memory_space=pltpu.SMEM
        ),
        out_shape=jax.ShapeDtypeStruct((1, 1), jnp.float32),
    )(tgt2d, g2d, pred)
    return out[0, 0]


# speedup vs baseline: 2.2282x; 2.2282x over previous
import functools
import math

import jax
import jax.numpy as jnp
import numpy as np
from jax.experimental import pallas as pl
from jax.experimental.pallas import tpu as pltpu

_SMOOTHING = 0.1
_BN = 32


def _loss_body(eps, coef_g, c_row, tgt_sref, tgt_ref, pred_ref, out_ref):
    i = pl.program_id(0)
    bn = pred_ref.shape[0]
    t = tgt_ref[...]
    x = pred_ref[...]
    valid = t != 0
    s = jnp.sum(x, axis=1, keepdims=True) - x[:, 0:1]
    part = jnp.sum(jnp.where(valid, s, 0.0))
    cnt = jnp.sum(jnp.where(valid, 1.0, 0.0))

    lane = jax.lax.broadcasted_iota(jnp.int32, (1, 128), 1)
    gpart = jnp.float32(0.0)
    for r in range(bn):
        tr = tgt_sref[i * bn + r]
        start = pl.multiple_of((tr // 128) * 128, 128)
        w = pred_ref[pl.ds(r, 1), pl.ds(start, 128)]
        gval = jnp.sum(jnp.where(lane == tr % 128, w, 0.0))
        gpart += jnp.where(tr != 0, gval, 0.0)

    @pl.when(i == 0)
    def _():
        out_ref[0, 0] = 0.0

    out_ref[0, 0] += c_row * cnt - eps * part - coef_g * gpart


def kernel(pred, target):
    n, v = pred.shape
    eps = _SMOOTHING / (v - 2)
    eps32 = float(np.float32(eps))
    c_row = (v - 2) * (eps32 * math.log(eps32)) + 0.9 * math.log(0.9)
    coef_g = (1.0 - _SMOOTHING) - eps

    tgt2d = target.reshape(n, 1)
    grid_spec = pltpu.PrefetchScalarGridSpec(
        num_scalar_prefetch=1,
        grid=(n // _BN,),
        in_specs=[
            pl.BlockSpec((_BN, 1), lambda i, *_: (i, 0)),
            pl.BlockSpec((_BN, v), lambda i, *_: (i, 0)),
        ],
        out_specs=pl.BlockSpec(
            (1, 1), lambda i, *_: (0, 0), memory_space=pltpu.SMEM
        ),
    )
    out = pl.pallas_call(
        functools.partial(_loss_body, eps, coef_g, c_row),
        grid_spec=grid_spec,
        out_shape=jax.ShapeDtypeStruct((1, 1), jnp.float32),
    )(target, tgt2d, pred)
    return out[0, 0]
